# baseline (speedup 1.0000x reference)
import functools

import jax
import jax.numpy as jnp
from jax import lax
from jax.experimental import pallas as pl
from jax.experimental.pallas import tpu as pltpu
from jax.experimental.pallas import tpu_sc as plsc

N_NODES = 50000
N_HEDGES = 50000
N_INC = 800000
N_GRAPHS = 256
H_DIM = 64
HEDGE_DIM = 35
NW = 32
D = 128

E_PAD = 802816
O_PAD = 102400


def _pad_idx(idx, n):
    return jnp.pad(idx.astype(jnp.int32), (0, n - idx.shape[0]))


def _pad_rows(w, rows=D):
    return jnp.pad(w, ((0, rows - w.shape[0]), (0, 0)))


@functools.partial(jax.jit, static_argnames=("ch",))
def _sc_gather(table, idx, ch=128):
    B = idx.shape[0]
    bpw = B // NW
    ch = min(ch, bpw)
    nch = bpw // ch
    mesh = plsc.VectorSubcoreMesh(core_axis_name="c", subcore_axis_name="s")

    @functools.partial(
        pl.kernel,
        mesh=mesh,
        out_type=jax.ShapeDtypeStruct((B, D), jnp.float32),
        scratch_types=[
            pltpu.VMEM((bpw,), jnp.int32),
            pltpu.VMEM((ch, D), jnp.float32),
            pltpu.SemaphoreType.DMA,
        ],
    )
    def k(table_hbm, idx_hbm, out_hbm, idx_v, rows_v, sem):
        wid = lax.axis_index("s") * 2 + lax.axis_index("c")
        base = wid * bpw
        pltpu.sync_copy(idx_hbm.at[pl.ds(base, bpw)], idx_v)

        def body(c, carry):
            o = pl.multiple_of(c * ch, 8)
            pltpu.async_copy(table_hbm.at[idx_v.at[pl.ds(o, ch)]], rows_v, sem).wait()
            pltpu.sync_copy(rows_v, out_hbm.at[pl.ds(base + o, ch)])
            return carry

        lax.fori_loop(0, nch, body, 0)

    return k(table, idx)


def _tril_strict(n):
    r = lax.broadcasted_iota(jnp.int32, (n, n), 0)
    c = lax.broadcasted_iota(jnp.int32, (n, n), 1)
    return (c < r).astype(jnp.float32)


def _cum_body(v, o_ref, carry_ref, comp_ref, i, s):
    @pl.when(i == 0)
    def _():
        carry_ref[...] = jnp.zeros_like(carry_ref)
        comp_ref[...] = jnp.zeros_like(comp_ref)

    blk = v.shape[0]
    g = blk // s
    t = _tril_strict(s)
    carry = carry_ref[...]
    off = jnp.zeros_like(carry)
    outs = []
    for j in range(g):
        seg = v[j * s : (j + 1) * s]
        cg = jnp.dot(t, seg, preferred_element_type=jnp.float32)
        outs.append(cg + (off + carry))
        off = off + jnp.sum(seg, 0, keepdims=True)
    o_ref[...] = jnp.concatenate(outs, 0)
    y = off - comp_ref[...]
    snew = carry + y
    comp_ref[...] = (snew - carry) - y
    carry_ref[...] = snew


def _cumsum_kernel(s, x_ref, o_ref, carry_ref, comp_ref):
    _cum_body(x_ref[...], o_ref, carry_ref, comp_ref, pl.program_id(0), s)


def _cumsum_excl(x, blk, s):
    n, d = x.shape
    return pl.pallas_call(
        functools.partial(_cumsum_kernel, s),
        grid=(n // blk,),
        in_specs=[pl.BlockSpec((blk, d), lambda i: (i, 0))],
        out_specs=pl.BlockSpec((blk, d), lambda i: (i, 0)),
        out_shape=jax.ShapeDtypeStruct((n, d), jnp.float32),
        scratch_shapes=[pltpu.VMEM((1, d), jnp.float32), pltpu.VMEM((1, d), jnp.float32)],
    )(x)


def _gate_cumsum_kernel(s, a_ref, b_ref, o_ref, carry_ref, comp_ref):
    z = a_ref[...] + b_ref[...]
    v = jax.nn.sigmoid(z[:, :H_DIM]) * jax.nn.softplus(z[:, H_DIM:])
    v = jnp.concatenate([v, jnp.zeros_like(v)], axis=1)
    _cum_body(v, o_ref, carry_ref, comp_ref, pl.program_id(0), s)


def _gate_cumsum_excl(a, b, blk, s):
    n, d = a.shape
    spec = pl.BlockSpec((blk, d), lambda i: (i, 0))
    return pl.pallas_call(
        functools.partial(_gate_cumsum_kernel, s),
        grid=(n // blk,),
        in_specs=[spec, spec],
        out_specs=spec,
        out_shape=jax.ShapeDtypeStruct((n, d), jnp.float32),
        scratch_shapes=[pltpu.VMEM((1, d), jnp.float32), pltpu.VMEM((1, d), jnp.float32)],
    )(a, b)


_BLK = 2000


def _full(r, c):
    return pl.BlockSpec((r, c), lambda i: (0, 0))


def _rows(c, blk=_BLK, off=0):
    return pl.BlockSpec((blk, c), lambda i: (i + off, 0))


def _embed_kernel(x_ref, w_ref, b_ref, o_ref):
    o_ref[...] = (
        jnp.dot(x_ref[...], w_ref[...], preferred_element_type=jnp.float32) + b_ref[...]
    )


def _embed(x0, w, b):
    return pl.pallas_call(
        _embed_kernel,
        grid=(N_NODES // _BLK,),
        in_specs=[_rows(92), _full(92, D), _full(1, D)],
        out_specs=_rows(D),
        out_shape=jax.ShapeDtypeStruct((N_NODES, D), jnp.float32),
    )(x0, w, b)


def _kAB_kernel(qhi, qlo, cnt, ha, wft, wfb, bf, wct, wcb, bc, zf_o, zc_o, st_o):
    i = pl.program_id(0)
    mx = (qhi[...] - qlo[...]) / cnt[...]
    h = ha[...]
    zf = (
        jnp.dot(mx, wft[...], preferred_element_type=jnp.float32)
        + jnp.dot(h, wfb[...], preferred_element_type=jnp.float32)
        + bf[...]
    )
    zc = (
        jnp.dot(mx, wct[...], preferred_element_type=jnp.float32)
        + jnp.dot(h, wcb[...], preferred_element_type=jnp.float32)
        + bc[...]
    )
    zf_o[...] = zf
    zc_o[...] = zc
    st = jnp.concatenate(
        [
            jnp.sum(zf, 0, keepdims=True),
            jnp.sum(zf * zf, 0, keepdims=True),
            jnp.sum(zc, 0, keepdims=True),
            jnp.sum(zc * zc, 0, keepdims=True),
            jnp.zeros((4, HEDGE_DIM), jnp.float32),
        ],
        axis=0,
    )

    @pl.when(i == 0)
    def _():
        st_o[...] = jnp.zeros_like(st_o)

    st_o[...] += st


def _kAB(qq, cnt, ha, wf1t, wc1t, p):
    return pl.pallas_call(
        _kAB_kernel,
        grid=(N_HEDGES // _BLK,),
        in_specs=[
            _rows(D), _rows(D, off=N_HEDGES // _BLK), _rows(1), _rows(HEDGE_DIM),
            _full(D, HEDGE_DIM), _full(HEDGE_DIM, HEDGE_DIM), _full(1, HEDGE_DIM),
            _full(D, HEDGE_DIM), _full(HEDGE_DIM, HEDGE_DIM), _full(1, HEDGE_DIM),
        ],
        out_specs=[_rows(HEDGE_DIM), _rows(HEDGE_DIM), _full(8, HEDGE_DIM)],
        out_shape=[
            jax.ShapeDtypeStruct((N_HEDGES, HEDGE_DIM), jnp.float32),
            jax.ShapeDtypeStruct((N_HEDGES, HEDGE_DIM), jnp.float32),
            jax.ShapeDtypeStruct((8, HEDGE_DIM), jnp.float32),
        ],
    )(
        qq, qq, cnt, ha,
        wf1t, p["w_f1"][H_DIM:], p["b_f1"].reshape(1, -1),
        wc1t, p["w_c1"][H_DIM:], p["b_c1"].reshape(1, -1),
    )


def _bn_from_stats(z, s0, s1, g, b, n):
    mu = s0 / n
    var = s1 / n - mu * mu
    return (z - mu) * lax.rsqrt(var + 1e-5) * g + b


def _kB2_kernel(zf, zc, st, gf, bf, gc, bc, x, w2ft, b2f, w2ct, b2c, w2fb, w2cb,
                ha_o, a_o, b_o):
    s = st[...]
    zfn = _bn_from_stats(zf[...], s[0:1], s[1:2], gf[...], bf[...], N_HEDGES)
    zcn = _bn_from_stats(zc[...], s[2:3], s[3:4], gc[...], bc[...], N_HEDGES)
    han = jax.nn.sigmoid(zfn) * jax.nn.softplus(zcn)
    ha_o[...] = han
    xb = x[...]
    af = jnp.dot(xb, w2ft[...], preferred_element_type=jnp.float32) + b2f[...]
    ac = jnp.dot(xb, w2ct[...], preferred_element_type=jnp.float32) + b2c[...]
    a_o[...] = jnp.concatenate([af, ac], axis=1)
    bfv = jnp.dot(han, w2fb[...], preferred_element_type=jnp.float32)
    bcv = jnp.dot(han, w2cb[...], preferred_element_type=jnp.float32)
    b_o[...] = jnp.concatenate([bfv, bcv], axis=1)


def _kB2(zf, zc, st, x, wf2t, wc2t, p):
    return pl.pallas_call(
        _kB2_kernel,
        grid=(N_HEDGES // _BLK,),
        in_specs=[
            _rows(HEDGE_DIM), _rows(HEDGE_DIM), _full(8, HEDGE_DIM),
            _full(1, HEDGE_DIM), _full(1, HEDGE_DIM), _full(1, HEDGE_DIM), _full(1, HEDGE_DIM),
            _rows(D),
            _full(D, H_DIM), _full(1, H_DIM), _full(D, H_DIM), _full(1, H_DIM),
            _full(HEDGE_DIM, H_DIM), _full(HEDGE_DIM, H_DIM),
        ],
        out_specs=[_rows(HEDGE_DIM), _rows(D), _rows(D)],
        out_shape=[
            jax.ShapeDtypeStruct((N_HEDGES, HEDGE_DIM), jnp.float32),
            jax.ShapeDtypeStruct((N_NODES, D), jnp.float32),
            jax.ShapeDtypeStruct((N_HEDGES, D), jnp.float32),
        ],
    )(
        zf, zc, st,
        p["bn_f_g"].reshape(1, -1), p["bn_f_b"].reshape(1, -1),
        p["bn_c_g"].reshape(1, -1), p["bn_c_b"].reshape(1, -1),
        x,
        wf2t, p["b_f2"].reshape(1, -1),
        wc2t, p["b_c2"].reshape(1, -1),
        p["w_f2"][H_DIM:], p["w_c2"][H_DIM:],
    )


def _kC1_kernel(qhi, qlo, cnt, o_ref, st_o):
    i = pl.program_id(0)
    out = (qhi[...] - qlo[...])[:, :H_DIM] / cnt[...]
    o_ref[...] = out
    st = jnp.concatenate(
        [
            jnp.sum(out, 0, keepdims=True),
            jnp.sum(out * out, 0, keepdims=True),
            jnp.zeros((6, H_DIM), jnp.float32),
        ],
        axis=0,
    )

    @pl.when(i == 0)
    def _():
        st_o[...] = jnp.zeros_like(st_o)

    st_o[...] += st


def _kC1(q2, cnt):
    return pl.pallas_call(
        _kC1_kernel,
        grid=(N_NODES // _BLK,),
        in_specs=[_rows(D), _rows(D, off=N_NODES // _BLK), _rows(1)],
        out_specs=[_rows(H_DIM), _full(8, H_DIM)],
        out_shape=[
            jax.ShapeDtypeStruct((N_NODES, H_DIM), jnp.float32),
            jax.ShapeDtypeStruct((8, H_DIM), jnp.float32),
        ],
    )(q2, q2, cnt)


def _kC2_kernel(o, st, g, b, x, xo):
    s = st[...]
    r = jax.nn.softplus(
        _bn_from_stats(o[...], s[0:1], s[1:2], g[...], b[...], N_NODES)
        + x[...][:, :H_DIM]
    )
    xo[...] = jnp.concatenate([r, jnp.zeros_like(r)], axis=1)


def _kC2(out, st, g, b, x):
    return pl.pallas_call(
        _kC2_kernel,
        grid=(N_NODES // _BLK,),
        in_specs=[_rows(H_DIM), _full(8, H_DIM), _full(1, H_DIM), _full(1, H_DIM), _rows(D)],
        out_specs=_rows(D),
        out_shape=jax.ShapeDtypeStruct((N_NODES, D), jnp.float32),
    )(out, st, g.reshape(1, -1), b.reshape(1, -1), x)


def _kfinal_kernel(phi, plo, cnt, l2w, l2b, ow, ob, o_ref):
    pooled = (phi[...] - plo[...]) / cnt[...]
    h = jax.nn.softplus(
        jnp.dot(pooled, l2w[...], preferred_element_type=jnp.float32) + l2b[...]
    )
    o_ref[...] = jnp.dot(h, ow[...], preferred_element_type=jnp.float32) + ob[...]


def _kfinal(phi, plo, cnt, l2wp, params):
    return pl.pallas_call(
        _kfinal_kernel,
        out_shape=jax.ShapeDtypeStruct((N_GRAPHS, 1), jnp.float32),
    )(
        phi, plo, cnt,
        l2wp, params["l2_b"].reshape(1, -1),
        params["out_w"], params["out_b"].reshape(1, -1),
    )


def kernel(x, hyperedge_index, hyperedge_attr, batch, params):
    src = hyperedge_index[0]
    he = hyperedge_index[1]

    perm_he = jnp.argsort(he)
    src_by_he = src[perm_he].astype(jnp.int32)
    he_sorted = he[perm_he]
    off_he = jnp.searchsorted(he_sorted, jnp.arange(N_HEDGES + 1)).astype(jnp.int32)
    perm_src = jnp.argsort(src)
    src_sorted = src[perm_src].astype(jnp.int32)
    he_by_src = he[perm_src].astype(jnp.int32)
    off_src = jnp.searchsorted(src_sorted, jnp.arange(N_NODES + 1)).astype(jnp.int32)
    off_b = jnp.searchsorted(batch, jnp.arange(N_GRAPHS + 1)).astype(jnp.int32)

    cnt_he = jnp.maximum(off_he[1:] - off_he[:-1], 1).astype(jnp.float32)[:, None]
    cnt_src = jnp.maximum(off_src[1:] - off_src[:-1], 1).astype(jnp.float32)[:, None]
    cnt_b = jnp.maximum(off_b[1:] - off_b[:-1], 1).astype(jnp.float32)[:, None]

    idx_e_he = _pad_idx(src_by_he, E_PAD)
    idx_q_he = _pad_idx(jnp.concatenate([off_he[1:], off_he[:-1]]), O_PAD)
    idx_a = _pad_idx(src_sorted, E_PAD)
    idx_b = _pad_idx(he_by_src, E_PAD)
    idx_q_src = _pad_idx(jnp.concatenate([off_src[1:], off_src[:-1]]), O_PAD)
    idx_q_b = jnp.concatenate([off_b[1:], off_b[:-1]]).astype(jnp.int32)

    embed_wp = jnp.pad(params["embed_w"], ((0, 0), (0, D - H_DIM)))
    embed_bp = jnp.pad(params["embed_b"], (0, D - H_DIM)).reshape(1, -1)
    l2wp = _pad_rows(params["l2_w"])

    x = _embed(x, embed_wp, embed_bp)
    ha = hyperedge_attr

    for p in params["convs"]:
        wf1t = _pad_rows(p["w_f1"][:H_DIM])
        wc1t = _pad_rows(p["w_c1"][:H_DIM])
        wf2t = _pad_rows(p["w_f2"][:H_DIM])
        wc2t = _pad_rows(p["w_c2"][:H_DIM])
        g = _sc_gather(x, idx_e_he)
        e1 = _cumsum_excl(g, 1024, 64)
        qq = _sc_gather(e1, idx_q_he)
        zf, zc, st = _kAB(qq, cnt_he, ha, wf1t, wc1t, p)
        ha, a_t, b_t = _kB2(zf, zc, st, x, wf2t, wc2t, p)
        ga = _sc_gather(a_t, idx_a)
        gb = _sc_gather(b_t, idx_b)
        e2 = _gate_cumsum_excl(ga, gb, 1024, 64)
        q2 = _sc_gather(e2, idx_q_src)
        out, st2 = _kC1(q2, cnt_src)
        x = _kC2(out, st2, p["bn_o_g"], p["bn_o_b"], x)

    xp = jnp.pad(x, ((0, 1000), (0, 0)))
    e3 = _cumsum_excl(xp, 1000, 40)
    q3 = _sc_gather(e3, idx_q_b, ch=16)
    return _kfinal(q3[:N_GRAPHS], q3[N_GRAPHS:], cnt_b, l2wp, params)
